# baseline (speedup 1.0000x reference)
import functools

import jax
import jax.numpy as jnp
from jax.experimental import pallas as pl
from jax.experimental.pallas import tpu as pltpu

B = 256
IN_DIM = 512
OUT_DIM = 10000
BN = 256


def _gcn_body(xt_ref, w_ref, adj_ref, bias_ref, out_ref, support_t_ref):
    @pl.when(pl.program_id(0) == 0)
    def _compute_support_t():
        st = jax.lax.dot_general(
            w_ref[...],
            xt_ref[...],
            dimension_numbers=(((0,), (0,)), ((), ())),
            preferred_element_type=jnp.float32,
        )
        support_t_ref[...] = st

    acc = jax.lax.dot_general(
        adj_ref[...],
        support_t_ref[...],
        dimension_numbers=(((1,), (0,)), ((), ())),
        preferred_element_type=jnp.float32,
        precision=jax.lax.Precision.DEFAULT,
    )
    out_ref[...] = acc.T + bias_ref[...]


@functools.partial(jax.jit, static_argnames=())
def kernel(input, adj, weight, bias):
    xt = input.T
    bias2d = bias.reshape(1, OUT_DIM)
    grid = (pl.cdiv(OUT_DIM, BN),)
    out = pl.pallas_call(
        _gcn_body,
        grid=grid,
        in_specs=[
            pl.BlockSpec((IN_DIM, B), lambda n: (0, 0)),
            pl.BlockSpec((IN_DIM, OUT_DIM), lambda n: (0, 0)),
            pl.BlockSpec((BN, OUT_DIM), lambda n: (n, 0)),
            pl.BlockSpec((1, BN), lambda n: (0, n)),
        ],
        out_specs=pl.BlockSpec((B, BN), lambda n: (0, n)),
        out_shape=jax.ShapeDtypeStruct((B, OUT_DIM), jnp.float32),
        scratch_shapes=[pltpu.VMEM((OUT_DIM, B), jnp.float32)],
        compiler_params=pltpu.CompilerParams(
            dimension_semantics=("arbitrary",),
        ),
    )(xt, weight, adj, bias2d)
    return out
